# baseline (speedup 1.0000x reference)
import functools

import jax
import jax.numpy as jnp
from jax import lax
from jax.experimental import pallas as pl
from jax.experimental.pallas import tpu as pltpu
from jax.experimental.pallas import tpu_sc as plsc

N = 10000
PER = 2500
EPER = 106666
HID = 128

NC = 2
NS = 16
NW = NC * NS

B = 96
NBUF = 5
NB = 35
NGROUP = NB // NBUF
EPAD = NW * NB * B
SLAB = 160
ACC_ROWS = NS * SLAB

_sc_mesh = plsc.VectorSubcoreMesh(
    core_axis_name="c", subcore_axis_name="s", num_cores=NC, num_subcores=NS
)


@functools.partial(
    pl.kernel,
    out_type=jax.ShapeDtypeStruct((NC, ACC_ROWS, HID), jnp.float32),
    mesh=_sc_mesh,
    scratch_types=[
        pltpu.VMEM((NB, B), jnp.int32),
        pltpu.VMEM((NB, B), jnp.int32),
    ]
    + [pltpu.VMEM((B, HID), jnp.float32) for _ in range(NBUF)]
    + [pltpu.SemaphoreType.DMA for _ in range(2 * NBUF)]
    + [
        pltpu.VMEM_SHARED((ACC_ROWS, HID), jnp.float32),
        pltpu.VMEM_SHARED((ACC_ROWS, HID), jnp.float32),
    ],
)
def _segment_sum_sc(table, sidx, didx, zeros, out, sidx_v, didx_v, *rest):
    rows = rest[:NBUF]
    gsem = rest[NBUF : 2 * NBUF]
    ssem = rest[2 * NBUF : 3 * NBUF]
    acc = rest[3 * NBUF]
    table_s = rest[3 * NBUF + 1]
    c = lax.axis_index("c")
    s = lax.axis_index("s")
    wid = c * NS + s
    pltpu.sync_copy(zeros.at[pl.ds(s * SLAB, SLAB)], acc.at[pl.ds(s * SLAB, SLAB)])

    pltpu.sync_copy(table.at[pl.ds(s * SLAB, SLAB)], table_s.at[pl.ds(s * SLAB, SLAB)])
    pltpu.sync_copy(sidx.at[wid], sidx_v)
    pltpu.sync_copy(didx.at[wid], didx_v)
    plsc.subcore_barrier()

    for b in range(NBUF):
        pltpu.async_copy(table_s.at[sidx_v.at[b]], rows[b], gsem[b])

    def group(g, carry):
        base = g * NBUF
        for b in range(NBUF):
            j = base + b
            pltpu.make_async_copy(table_s.at[sidx_v.at[j]], rows[b], gsem[b]).wait()
            pltpu.async_copy(rows[b], acc.at[didx_v.at[j]], ssem[b], add=True)
        for b in range(NBUF):
            j = base + b
            jn = jnp.minimum(j + NBUF, NB - 1)
            pltpu.make_async_copy(rows[b], acc.at[didx_v.at[j]], ssem[b]).wait()
            pltpu.async_copy(table_s.at[sidx_v.at[jn]], rows[b], gsem[b])
        return carry

    lax.fori_loop(0, NGROUP, group, 0)
    for b in range(NBUF):
        pltpu.make_async_copy(table_s.at[sidx_v.at[NB - 1]], rows[b], gsem[b]).wait()
    plsc.subcore_barrier()
    pltpu.sync_copy(acc.at[pl.ds(s * SLAB, SLAB)], out.at[c, pl.ds(s * SLAB, SLAB)])


def _embed_body(x_ref, w_ref, b_ref, o_ref):
    o_ref[...] = jnp.tanh(
        jnp.dot(x_ref[...], w_ref[...], preferred_element_type=jnp.float32)
        + b_ref[...]
    )


def _resnet(x, w):
    h1 = jnp.tanh(jnp.dot(x, w[0], preferred_element_type=jnp.float32) + w[1])
    h2 = jnp.tanh(jnp.dot(h1, w[2], preferred_element_type=jnp.float32) + w[3])
    return jnp.dot(h2 + x, w[4], preferred_element_type=jnp.float32) + w[5]


def _level_body(*refs):
    p, ope = refs[0], refs[1]
    w = [r[...] for r in refs[2:26]]
    o = refs[26]
    ms = p[0, :PER, :] + p[1, :PER, :]
    mr = jnp.tanh(_resnet(ms, w[0:6]))
    mr = jnp.tanh(_resnet(mr, w[6:12]))
    cc = jnp.concatenate([ope[...], mr], axis=-1)
    e = jnp.tanh(_resnet(cc, w[12:18]))
    e = jnp.tanh(_resnet(e, w[18:24]))
    o[...] = jnp.concatenate(
        [e, jnp.zeros((ACC_ROWS - PER, HID), jnp.float32)], axis=0
    )


def _level_mlp(part, ope_l, wlist):
    return pl.pallas_call(
        _level_body,
        out_shape=jax.ShapeDtypeStruct((ACC_ROWS, HID), jnp.float32),
    )(part, ope_l, *wlist)


def kernel(node_feats, edge_index, node_depth, params):
    eW = params["embed_W"]
    eb = params["embed_b"].reshape(1, HID)
    embeds = pl.pallas_call(
        _embed_body,
        out_shape=jax.ShapeDtypeStruct((N, HID), jnp.float32),
    )(node_feats, eW, eb)

    zeros = jnp.zeros((ACC_ROWS, HID), jnp.float32)
    pad_s = jnp.arange(EPAD - EPER, dtype=jnp.int32) % PER
    pad_t = PER + jnp.arange(EPAD - EPER, dtype=jnp.int32) % (ACC_ROWS - PER)

    table = embeds
    out = embeds
    for l in (1, 2, 3):
        d = min(l - 1, 2)
        s_rel = edge_index[0, (l - 1) * EPER : l * EPER] - (l - 1) * PER
        t_rel = edge_index[1, (l - 1) * EPER : l * EPER] - l * PER
        sidx = jnp.concatenate([s_rel, pad_s]).reshape(NW, NB, B)
        didx = jnp.concatenate([t_rel, pad_t]).reshape(NW, NB, B)
        part = _segment_sum_sc(table, sidx, didx, zeros)
        ope_l = lax.slice(embeds, (l * PER, 0), ((l + 1) * PER, HID))
        wlist = []
        for rp in (
            params["mp_bwd"][d],
            params["mp_bwd1"][d],
            params["node_embeds"][d],
            params["node_embeds1"][d],
        ):
            wlist += [
                rp["W1"], rp["b1"].reshape(1, -1),
                rp["W2"], rp["b2"].reshape(1, -1),
                rp["W3"], rp["b3"].reshape(1, -1),
            ]
        e = _level_mlp(part, ope_l, wlist)
        table = e
        e_sl = lax.slice(e, (0, 0), (PER, HID))
        out = lax.dynamic_update_slice(out, e_sl, (l * PER, 0))
    return out


# speedup vs baseline: 1.1335x; 1.1335x over previous
import functools

import jax
import jax.numpy as jnp
from jax import lax
from jax.experimental import pallas as pl
from jax.experimental.pallas import tpu as pltpu
from jax.experimental.pallas import tpu_sc as plsc

N = 10000
PER = 2500
EPER = 106666
HID = 128

NC = 2
NS = 16
NW = NC * NS

B = 96
NBUF = 6
NB = 36
NGROUP = NB // NBUF
EPAD = NW * NB * B
SLAB = 160
ACC_ROWS = NS * SLAB

_sc_mesh = plsc.VectorSubcoreMesh(
    core_axis_name="c", subcore_axis_name="s", num_cores=NC, num_subcores=NS
)


@functools.partial(
    pl.kernel,
    out_type=jax.ShapeDtypeStruct((NC, ACC_ROWS, HID), jnp.float32),
    mesh=_sc_mesh,
    scratch_types=[
        pltpu.VMEM((NB, B), jnp.int32),
        pltpu.VMEM((NB, B), jnp.int32),
    ]
    + [pltpu.VMEM((B, HID), jnp.float32) for _ in range(NBUF)]
    + [pltpu.SemaphoreType.DMA for _ in range(2 * NBUF)]
    + [pltpu.VMEM_SHARED((ACC_ROWS, HID), jnp.float32)],
)
def _segment_sum_sc(table, sidx, didx, zeros, out, sidx_v, didx_v, *rest):
    rows = rest[:NBUF]
    gsem = rest[NBUF : 2 * NBUF]
    ssem = rest[2 * NBUF : 3 * NBUF]
    acc = rest[3 * NBUF]
    c = lax.axis_index("c")
    s = lax.axis_index("s")
    wid = c * NS + s
    pltpu.sync_copy(sidx.at[wid], sidx_v)
    for b in range(NBUF):
        pltpu.async_copy(table.at[sidx_v.at[b]], rows[b], gsem[b])
    pltpu.sync_copy(zeros.at[pl.ds(s * SLAB, SLAB)], acc.at[pl.ds(s * SLAB, SLAB)])
    pltpu.sync_copy(didx.at[wid], didx_v)
    plsc.subcore_barrier()


    def group(g, carry):
        base = g * NBUF
        for b in range(NBUF):
            j = base + b
            pltpu.make_async_copy(table.at[sidx_v.at[j]], rows[b], gsem[b]).wait()
            pltpu.async_copy(rows[b], acc.at[didx_v.at[j]], ssem[b], add=True)
        for b in range(NBUF):
            j = base + b
            jn = jnp.minimum(j + NBUF, NB - 1)
            pltpu.make_async_copy(rows[b], acc.at[didx_v.at[j]], ssem[b]).wait()
            pltpu.async_copy(table.at[sidx_v.at[jn]], rows[b], gsem[b])
        return carry

    lax.fori_loop(0, NGROUP, group, 0)
    for b in range(NBUF):
        pltpu.make_async_copy(table.at[sidx_v.at[NB - 1]], rows[b], gsem[b]).wait()
    plsc.subcore_barrier()
    pltpu.sync_copy(acc.at[pl.ds(s * SLAB, SLAB)], out.at[c, pl.ds(s * SLAB, SLAB)])


def _embed_body(x_ref, w_ref, b_ref, o_ref):
    o_ref[...] = jnp.tanh(
        jnp.dot(x_ref[...], w_ref[...], preferred_element_type=jnp.float32)
        + b_ref[...]
    )


def _resnet(x, w):
    h1 = jnp.tanh(jnp.dot(x, w[0], preferred_element_type=jnp.float32) + w[1])
    h2 = jnp.tanh(jnp.dot(h1, w[2], preferred_element_type=jnp.float32) + w[3])
    return jnp.dot(h2 + x, w[4], preferred_element_type=jnp.float32) + w[5]


def _level_body(*refs):
    p, ope = refs[0], refs[1]
    w = [r[...] for r in refs[2:26]]
    o = refs[26]
    ms = p[0, :PER, :] + p[1, :PER, :]
    mr = jnp.tanh(_resnet(ms, w[0:6]))
    mr = jnp.tanh(_resnet(mr, w[6:12]))
    cc = jnp.concatenate([ope[...], mr], axis=-1)
    e = jnp.tanh(_resnet(cc, w[12:18]))
    e = jnp.tanh(_resnet(e, w[18:24]))
    o[...] = jnp.concatenate(
        [e, jnp.zeros((ACC_ROWS - PER, HID), jnp.float32)], axis=0
    )


def _level_mlp(part, ope_l, wlist):
    return pl.pallas_call(
        _level_body,
        out_shape=jax.ShapeDtypeStruct((ACC_ROWS, HID), jnp.float32),
    )(part, ope_l, *wlist)


def kernel(node_feats, edge_index, node_depth, params):
    eW = params["embed_W"]
    eb = params["embed_b"].reshape(1, HID)
    embeds = pl.pallas_call(
        _embed_body,
        out_shape=jax.ShapeDtypeStruct((N, HID), jnp.float32),
    )(node_feats, eW, eb)

    zeros = jnp.zeros((ACC_ROWS, HID), jnp.float32)
    pad_s = jnp.arange(EPAD - EPER, dtype=jnp.int32) % PER
    pad_t = PER + jnp.arange(EPAD - EPER, dtype=jnp.int32) % (ACC_ROWS - PER)

    table = embeds
    out = embeds
    for l in (1, 2, 3):
        d = min(l - 1, 2)
        s_rel = edge_index[0, (l - 1) * EPER : l * EPER] - (l - 1) * PER
        t_rel = edge_index[1, (l - 1) * EPER : l * EPER] - l * PER
        sidx = jnp.concatenate([s_rel, pad_s]).reshape(NW, NB, B)
        didx = jnp.concatenate([t_rel, pad_t]).reshape(NW, NB, B)
        part = _segment_sum_sc(table, sidx, didx, zeros)
        ope_l = lax.slice(embeds, (l * PER, 0), ((l + 1) * PER, HID))
        wlist = []
        for rp in (
            params["mp_bwd"][d],
            params["mp_bwd1"][d],
            params["node_embeds"][d],
            params["node_embeds1"][d],
        ):
            wlist += [
                rp["W1"], rp["b1"].reshape(1, -1),
                rp["W2"], rp["b2"].reshape(1, -1),
                rp["W3"], rp["b3"].reshape(1, -1),
            ]
        e = _level_mlp(part, ope_l, wlist)
        table = e
        e_sl = lax.slice(e, (0, 0), (PER, HID))
        out = lax.dynamic_update_slice(out, e_sl, (l * PER, 0))
    return out


# speedup vs baseline: 1.1527x; 1.0170x over previous
import functools

import jax
import jax.numpy as jnp
from jax import lax
from jax.experimental import pallas as pl
from jax.experimental.pallas import tpu as pltpu
from jax.experimental.pallas import tpu_sc as plsc

N = 10000
PER = 2500
EPER = 106666
HID = 128

NC = 2
NS = 16
NW = NC * NS

B = 96
NBUF = 6
NB = 36
NGROUP = NB // NBUF
EPAD = NW * NB * B
SLAB = 160
ACC_ROWS = NS * SLAB

_sc_mesh = plsc.VectorSubcoreMesh(
    core_axis_name="c", subcore_axis_name="s", num_cores=NC, num_subcores=NS
)


@functools.partial(
    pl.kernel,
    out_type=jax.ShapeDtypeStruct((NC, ACC_ROWS, HID), jnp.float32),
    mesh=_sc_mesh,
    scratch_types=[
        pltpu.VMEM((NB, B), jnp.int32),
        pltpu.VMEM((NB, B), jnp.int32),
    ]
    + [pltpu.VMEM((B, HID), jnp.float32) for _ in range(NBUF)]
    + [pltpu.SemaphoreType.DMA for _ in range(2 * NBUF)]
    + [pltpu.VMEM_SHARED((ACC_ROWS, HID), jnp.float32)],
)
def _segment_sum_sc(table, sidx, didx, zeros, out, sidx_v, didx_v, *rest):
    rows = rest[:NBUF]
    gsem = rest[NBUF : 2 * NBUF]
    ssem = rest[2 * NBUF : 3 * NBUF]
    acc = rest[3 * NBUF]
    c = lax.axis_index("c")
    s = lax.axis_index("s")
    wid = c * NS + s
    pltpu.sync_copy(sidx.at[wid], sidx_v)
    for b in range(NBUF):
        pltpu.async_copy(table.at[sidx_v.at[b]], rows[b], gsem[b])
    pltpu.sync_copy(zeros.at[pl.ds(s * SLAB, SLAB)], acc.at[pl.ds(s * SLAB, SLAB)])
    pltpu.sync_copy(didx.at[wid], didx_v)
    plsc.subcore_barrier()


    def group(g, carry):
        base = g * NBUF
        for b in range(NBUF):
            j = base + b
            pltpu.make_async_copy(table.at[sidx_v.at[j]], rows[b], gsem[b]).wait()
            pltpu.async_copy(rows[b], acc.at[didx_v.at[j]], ssem[b], add=True)
        for b in range(NBUF):
            j = base + b
            jn = jnp.minimum(j + NBUF, NB - 1)
            pltpu.make_async_copy(rows[b], acc.at[didx_v.at[j]], ssem[b]).wait()
            pltpu.async_copy(table.at[sidx_v.at[jn]], rows[b], gsem[b])
        return carry

    lax.fori_loop(0, NGROUP, group, 0)
    for b in range(NBUF):
        pltpu.make_async_copy(table.at[sidx_v.at[NB - 1]], rows[b], gsem[b]).wait()
    plsc.subcore_barrier()
    pltpu.sync_copy(acc.at[pl.ds(s * SLAB, SLAB)], out.at[c, pl.ds(s * SLAB, SLAB)])


def _embed_body(x_ref, w_ref, b_ref, o_ref):
    o_ref[...] = jnp.tanh(
        jnp.dot(x_ref[...], w_ref[...], preferred_element_type=jnp.float32)
        + b_ref[...]
    )


def _resnet(x, w):
    h1 = jnp.tanh(jnp.dot(x, w[0], preferred_element_type=jnp.float32) + w[1])
    h2 = jnp.tanh(jnp.dot(h1, w[2], preferred_element_type=jnp.float32) + w[3])
    return jnp.dot(h2 + x, w[4], preferred_element_type=jnp.float32) + w[5]


def _level_body(*refs):
    p, ope = refs[0], refs[1]
    w = [r[...] for r in refs[2:26]]
    o = refs[26]
    ms = p[0, :PER, :] + p[1, :PER, :]
    mr = jnp.tanh(_resnet(ms, w[0:6]))
    mr = jnp.tanh(_resnet(mr, w[6:12]))
    cc = jnp.concatenate([ope[...], mr], axis=-1)
    e = jnp.tanh(_resnet(cc, w[12:18]))
    o[...] = jnp.tanh(_resnet(e, w[18:24]))


def _level_mlp(part, ope_l, wlist):
    return pl.pallas_call(
        _level_body,
        out_shape=jax.ShapeDtypeStruct((PER, HID), jnp.float32),
    )(part, ope_l, *wlist)


def kernel(node_feats, edge_index, node_depth, params):
    eW = params["embed_W"]
    eb = params["embed_b"].reshape(1, HID)
    embeds = pl.pallas_call(
        _embed_body,
        out_shape=jax.ShapeDtypeStruct((N, HID), jnp.float32),
    )(node_feats, eW, eb)

    zeros = jnp.zeros((ACC_ROWS, HID), jnp.float32)
    pad_s = jnp.arange(EPAD - EPER, dtype=jnp.int32) % PER
    pad_t = PER + jnp.arange(EPAD - EPER, dtype=jnp.int32) % (ACC_ROWS - PER)

    table = embeds
    out = embeds
    for l in (1, 2, 3):
        d = min(l - 1, 2)
        s_rel = edge_index[0, (l - 1) * EPER : l * EPER] - (l - 1) * PER
        t_rel = edge_index[1, (l - 1) * EPER : l * EPER] - l * PER
        sidx = jnp.concatenate([s_rel, pad_s]).reshape(NW, NB, B)
        didx = jnp.concatenate([t_rel, pad_t]).reshape(NW, NB, B)
        part = _segment_sum_sc(table, sidx, didx, zeros)
        ope_l = lax.slice(embeds, (l * PER, 0), ((l + 1) * PER, HID))
        wlist = []
        for rp in (
            params["mp_bwd"][d],
            params["mp_bwd1"][d],
            params["node_embeds"][d],
            params["node_embeds1"][d],
        ):
            wlist += [
                rp["W1"], rp["b1"].reshape(1, -1),
                rp["W2"], rp["b2"].reshape(1, -1),
                rp["W3"], rp["b3"].reshape(1, -1),
            ]
        e = _level_mlp(part, ope_l, wlist)
        table = e
        out = lax.dynamic_update_slice(out, e, (l * PER, 0))
    return out
